# baseline (speedup 1.0000x reference)
import functools

import jax
import jax.numpy as jnp
from jax import lax
from jax.experimental import pallas as pl
from jax.experimental.pallas import tpu as pltpu
from jax.experimental.pallas import tpu_sc as plsc

C_LIGHT = 299792.458
CENTER_FREQ = 437100000.0

N = 1_000_000
N_PASSES = 100_000
NW = 32
CHUNK = 4096
NFULL = 7
PER_W = 31_264
TAIL = 2592
VPC = CHUNK // 16
NBANK = 8
NOUT = 2
TBLP = 50_008
DELTA_W = N_PASSES // 2

_mesh = plsc.VectorSubcoreMesh(core_axis_name="c", subcore_axis_name="s")

_scratch = (
    [pltpu.VMEM_SHARED((TBLP,), jnp.int32),
     pltpu.VMEM((TBLP,), jnp.int32)]
    + [pltpu.VMEM((CHUNK,), jnp.int32) for _ in range(NBANK)]
    + [pltpu.VMEM((CHUNK,), jnp.float32) for _ in range(NBANK)]
    + [pltpu.VMEM((CHUNK,), jnp.float32) for _ in range(NOUT)]
    + [pltpu.SemaphoreType.DMA for _ in range(NBANK + NOUT)]
)


@functools.partial(
    pl.kernel,
    out_type=jax.ShapeDtypeStruct((N,), jnp.float32),
    mesh=_mesh,
    compiler_params=pltpu.CompilerParams(needs_layout_passes=False),
    scratch_types=_scratch,
)
def _doppler_sc(rr_hbm, packed_hbm, idx_hbm, out_hbm, table_sh, table_v, *bufs):
    idx_b = bufs[:NBANK]
    rr_b = bufs[NBANK:2 * NBANK]
    out_b = bufs[2 * NBANK:2 * NBANK + NOUT]
    si_b = bufs[2 * NBANK + NOUT:2 * NBANK + NOUT + NBANK]
    so_b = bufs[2 * NBANK + NOUT + NBANK:]

    wid = lax.axis_index("s") * 2 + lax.axis_index("c")
    base = jnp.minimum(wid * PER_W, N - PER_W)

    def start_in(g, size=CHUNK):
        off = base + g * CHUNK
        pltpu.make_async_copy(
            idx_hbm.at[pl.ds(off, size)], idx_b[g].at[pl.ds(0, size)],
            si_b[g]).start()
        pltpu.make_async_copy(
            rr_hbm.at[pl.ds(off, size)], rr_b[g].at[pl.ds(0, size)],
            si_b[g]).start()

    def wait_in(g, size=CHUNK):
        pltpu.make_async_copy(
            idx_hbm.at[pl.ds(base, size)], idx_b[g].at[pl.ds(0, size)],
            si_b[g]).wait()
        pltpu.make_async_copy(
            rr_hbm.at[pl.ds(base, size)], rr_b[g].at[pl.ds(0, size)],
            si_b[g]).wait()

    def start_out(g, size=CHUNK):
        off = base + g * CHUNK
        pltpu.make_async_copy(
            out_b[g % NOUT].at[pl.ds(0, size)], out_hbm.at[pl.ds(off, size)],
            so_b[g % NOUT]).start()

    def wait_out(g, size=CHUNK):
        pltpu.make_async_copy(
            out_b[g % NOUT].at[pl.ds(0, size)], out_hbm.at[pl.ds(base, size)],
            so_b[g % NOUT]).wait()

    for g in range(NFULL):
        start_in(g)
    start_in(NFULL, TAIL)

    @pl.when(lax.axis_index("s") == 0)
    def _():
        pltpu.sync_copy(packed_hbm, table_sh)

    plsc.subcore_barrier()
    pltpu.sync_copy(table_sh, table_v)

    def unpack(word, half):
        shift = (1 - half) << 4
        bits = (word << shift) & jnp.int32(-65536)
        return plsc.bitcast(bits, jnp.float32)

    dword = plsc.load_gather(table_v, [jnp.full((16,), DELTA_W, jnp.int32)])
    delta = unpack(dword, jnp.zeros((16,), jnp.int32))
    scale = -(CENTER_FREQ + delta) / C_LIGHT

    def compute(g, nvec, unroll):
        ib, rb, ob = idx_b[g], rr_b[g], out_b[g % NOUT]

        def step(i):
            sl = pl.ds(pl.multiple_of(i * 16, 16), 16)
            iv = ib[sl]
            word = plsc.load_gather(table_v, [iv >> 1])
            bias = unpack(word, iv & 1)
            ob[sl] = rb[sl] * scale + bias

        plsc.parallel_loop(0, nvec, 1, unroll=unroll)(step)

    for g in range(NFULL):
        wait_in(g)
        if g >= NOUT:
            wait_out(g)
        compute(g, VPC, 8)
        start_out(g)

    wait_in(NFULL, TAIL)
    wait_out(NFULL)
    compute(NFULL, TAIL // 16, 2)
    start_out(NFULL, TAIL)
    wait_out(NFULL - 1)
    wait_out(NFULL, TAIL)


def kernel(range_rate, sensor_params, contact_indices):
    idx32 = contact_indices.astype(jnp.int32)
    pv = sensor_params.astype(jnp.bfloat16)
    u = lax.bitcast_convert_type(pv, jnp.uint16).astype(jnp.uint32)
    u = jnp.pad(u, (0, 2 * TBLP - (N_PASSES + 1)))
    pairs = u.reshape(TBLP, 2)
    packed = lax.bitcast_convert_type(
        pairs[:, 0] | (pairs[:, 1] << 16), jnp.int32)
    return _doppler_sc(range_rate, packed, idx32)


# speedup vs baseline: 1.7760x; 1.7760x over previous
import functools

import jax
import jax.numpy as jnp
from jax import lax
from jax.experimental import pallas as pl
from jax.experimental.pallas import tpu as pltpu
from jax.experimental.pallas import tpu_sc as plsc

C_LIGHT = 299792.458
CENTER_FREQ = 437100000.0

N = 1_000_000
N_PASSES = 100_000
NW = 32
CHUNK = 4096
NFULL = 7
PER_W = 31_264
TAIL = 2592
VPC = CHUNK // 16
NBANK = 8
NOUT = 2
TBLP = 50_008
DELTA_W = N_PASSES // 2

_mesh = plsc.VectorSubcoreMesh(core_axis_name="c", subcore_axis_name="s")

_scratch = (
    [pltpu.VMEM_SHARED((TBLP,), jnp.int32),
     pltpu.VMEM((TBLP,), jnp.int32)]
    + [pltpu.VMEM((CHUNK,), jnp.int32) for _ in range(NBANK)]
    + [pltpu.VMEM((CHUNK,), jnp.float32) for _ in range(NBANK)]
    + [pltpu.VMEM((CHUNK,), jnp.float32) for _ in range(NOUT)]
    + [pltpu.SemaphoreType.DMA for _ in range(NBANK + NOUT)]
)


@functools.partial(
    pl.kernel,
    out_type=jax.ShapeDtypeStruct((N,), jnp.float32),
    mesh=_mesh,
    compiler_params=pltpu.CompilerParams(needs_layout_passes=False),
    scratch_types=_scratch,
)
def _doppler_sc(rr_hbm, packed_hbm, idx_hbm, out_hbm, table_sh, table_v, *bufs):
    idx_b = bufs[:NBANK]
    rr_b = bufs[NBANK:2 * NBANK]
    out_b = bufs[2 * NBANK:2 * NBANK + NOUT]
    si_b = bufs[2 * NBANK + NOUT:2 * NBANK + NOUT + NBANK]
    so_b = bufs[2 * NBANK + NOUT + NBANK:]

    wid = lax.axis_index("s") * 2 + lax.axis_index("c")
    base = jnp.minimum(wid * PER_W, N - PER_W)

    def start_in(g, size=CHUNK):
        off = base + g * CHUNK
        pltpu.make_async_copy(
            idx_hbm.at[pl.ds(off, size)], idx_b[g].at[pl.ds(0, size)],
            si_b[g]).start()
        pltpu.make_async_copy(
            rr_hbm.at[pl.ds(off, size)], rr_b[g].at[pl.ds(0, size)],
            si_b[g]).start()

    def wait_in(g, size=CHUNK):
        pltpu.make_async_copy(
            idx_hbm.at[pl.ds(base, size)], idx_b[g].at[pl.ds(0, size)],
            si_b[g]).wait()
        pltpu.make_async_copy(
            rr_hbm.at[pl.ds(base, size)], rr_b[g].at[pl.ds(0, size)],
            si_b[g]).wait()

    def start_out(g, size=CHUNK):
        off = base + g * CHUNK
        pltpu.make_async_copy(
            out_b[g % NOUT].at[pl.ds(0, size)], out_hbm.at[pl.ds(off, size)],
            so_b[g % NOUT]).start()

    def wait_out(g, size=CHUNK):
        pltpu.make_async_copy(
            out_b[g % NOUT].at[pl.ds(0, size)], out_hbm.at[pl.ds(base, size)],
            so_b[g % NOUT]).wait()

    for g in range(NFULL):
        start_in(g)
    start_in(NFULL, TAIL)

    @pl.when(lax.axis_index("s") == 0)
    def _():
        pltpu.sync_copy(packed_hbm, table_sh)

    plsc.subcore_barrier()
    pltpu.sync_copy(table_sh, table_v)

    def unpack(word, half):
        shift = (1 - half) << 4
        bits = (word << shift) & jnp.int32(-65536)
        return plsc.bitcast(bits, jnp.float32)

    dword = plsc.load_gather(table_v, [jnp.full((16,), DELTA_W, jnp.int32)])
    delta = unpack(dword, jnp.zeros((16,), jnp.int32))
    scale = -(CENTER_FREQ + delta) / C_LIGHT

    def compute(g, nvec, unroll):
        ib, rb, ob = idx_b[g], rr_b[g], out_b[g % NOUT]

        def step(i):
            sl = pl.ds(pl.multiple_of(i * 16, 16), 16)
            iv = ib[sl]
            word = plsc.load_gather(table_v, [iv >> 1])
            bias = unpack(word, iv & 1)
            ob[sl] = rb[sl] * scale + bias

        plsc.parallel_loop(0, nvec, 1, unroll=unroll)(step)

    for g in range(NFULL):
        wait_in(g)
        if g >= NOUT:
            wait_out(g)
        compute(g, VPC, 8)
        start_out(g)

    wait_in(NFULL, TAIL)
    wait_out(NFULL)
    compute(NFULL, TAIL // 16, 2)
    start_out(NFULL, TAIL)
    wait_out(NFULL - 1)
    wait_out(NFULL, TAIL)


def kernel(range_rate, sensor_params, contact_indices):
    idx32 = contact_indices.astype(jnp.int32)
    xp = jnp.pad(sensor_params, (0, 2 * TBLP - (N_PASSES + 1)))
    xi = lax.bitcast_convert_type(xp, jnp.int32)
    b = ((xi + 0x7FFF + ((xi >> 16) & 1)) >> 16) & 0xFFFF
    packed = b[0::2] | (b[1::2] << 16)
    return _doppler_sc(range_rate, packed, idx32)


# speedup vs baseline: 2.8188x; 1.5871x over previous
import functools

import jax
import jax.numpy as jnp
from jax import lax
from jax.experimental import pallas as pl
from jax.experimental.pallas import tpu as pltpu
from jax.experimental.pallas import tpu_sc as plsc

C_LIGHT = 299792.458
CENTER_FREQ = 437100000.0

N = 1_000_000
N_PASSES = 100_000
NW = 32
CHUNK = 4096
NFULL = 7
PER_W = 31_264
TAIL = 2592
VPC = CHUNK // 16
NBANK = 8
NOUT = 2
TBLP = 50_001
DELTA_W = N_PASSES - TBLP

_mesh = plsc.VectorSubcoreMesh(core_axis_name="c", subcore_axis_name="s")

_scratch = (
    [pltpu.VMEM_SHARED((TBLP,), jnp.int32),
     pltpu.VMEM((TBLP,), jnp.int32)]
    + [pltpu.VMEM((CHUNK,), jnp.int32) for _ in range(NBANK)]
    + [pltpu.VMEM((CHUNK,), jnp.float32) for _ in range(NBANK)]
    + [pltpu.VMEM((CHUNK,), jnp.float32) for _ in range(NOUT)]
    + [pltpu.SemaphoreType.DMA for _ in range(NBANK + NOUT)]
)


@functools.partial(
    pl.kernel,
    out_type=jax.ShapeDtypeStruct((N,), jnp.float32),
    mesh=_mesh,
    compiler_params=pltpu.CompilerParams(needs_layout_passes=False),
    scratch_types=_scratch,
)
def _doppler_sc(rr_hbm, packed_hbm, idx_hbm, out_hbm, table_sh, table_v, *bufs):
    idx_b = bufs[:NBANK]
    rr_b = bufs[NBANK:2 * NBANK]
    out_b = bufs[2 * NBANK:2 * NBANK + NOUT]
    si_b = bufs[2 * NBANK + NOUT:2 * NBANK + NOUT + NBANK]
    so_b = bufs[2 * NBANK + NOUT + NBANK:]

    wid = lax.axis_index("s") * 2 + lax.axis_index("c")
    base = jnp.minimum(wid * PER_W, N - PER_W)

    def start_in(g, size=CHUNK):
        off = base + g * CHUNK
        pltpu.make_async_copy(
            idx_hbm.at[pl.ds(off, size)], idx_b[g].at[pl.ds(0, size)],
            si_b[g]).start()
        pltpu.make_async_copy(
            rr_hbm.at[pl.ds(off, size)], rr_b[g].at[pl.ds(0, size)],
            si_b[g]).start()

    def wait_in(g, size=CHUNK):
        pltpu.make_async_copy(
            idx_hbm.at[pl.ds(base, size)], idx_b[g].at[pl.ds(0, size)],
            si_b[g]).wait()
        pltpu.make_async_copy(
            rr_hbm.at[pl.ds(base, size)], rr_b[g].at[pl.ds(0, size)],
            si_b[g]).wait()

    def start_out(g, size=CHUNK):
        off = base + g * CHUNK
        pltpu.make_async_copy(
            out_b[g % NOUT].at[pl.ds(0, size)], out_hbm.at[pl.ds(off, size)],
            so_b[g % NOUT]).start()

    def wait_out(g, size=CHUNK):
        pltpu.make_async_copy(
            out_b[g % NOUT].at[pl.ds(0, size)], out_hbm.at[pl.ds(base, size)],
            so_b[g % NOUT]).wait()

    for g in range(NFULL):
        start_in(g)
    start_in(NFULL, TAIL)

    @pl.when(lax.axis_index("s") == 0)
    def _():
        pltpu.sync_copy(packed_hbm, table_sh)

    plsc.subcore_barrier()
    pltpu.sync_copy(table_sh, table_v)

    def unpack(word, half):
        shift = (1 - half) << 4
        bits = (word << shift) & jnp.int32(-65536)
        return plsc.bitcast(bits, jnp.float32)

    dword = plsc.load_gather(table_v, [jnp.full((16,), DELTA_W, jnp.int32)])
    delta = unpack(dword, jnp.ones((16,), jnp.int32))
    scale = -(CENTER_FREQ + delta) / C_LIGHT

    def compute(g, nvec, unroll):
        ib, rb, ob = idx_b[g], rr_b[g], out_b[g % NOUT]

        def step(i):
            sl = pl.ds(pl.multiple_of(i * 16, 16), 16)
            iv = ib[sl]
            half = (iv >= TBLP).astype(jnp.int32)
            word = plsc.load_gather(table_v, [iv - half * TBLP])
            bias = unpack(word, half)
            ob[sl] = rb[sl] * scale + bias

        plsc.parallel_loop(0, nvec, 1, unroll=unroll)(step)

    for g in range(NFULL):
        wait_in(g)
        if g >= NOUT:
            wait_out(g)
        compute(g, VPC, 8)
        start_out(g)

    wait_in(NFULL, TAIL)
    wait_out(NFULL)
    compute(NFULL, TAIL // 16, 2)
    start_out(NFULL, TAIL)
    wait_out(NFULL - 1)
    wait_out(NFULL, TAIL)


def kernel(range_rate, sensor_params, contact_indices):
    idx32 = contact_indices.astype(jnp.int32)
    xp = jnp.pad(sensor_params, (0, 2 * TBLP - (N_PASSES + 1)))
    xi = lax.bitcast_convert_type(xp, jnp.int32)
    b = ((xi + 0x7FFF + ((xi >> 16) & 1)) >> 16) & 0xFFFF
    packed = b[:TBLP] | (b[TBLP:] << 16)
    return _doppler_sc(range_rate, packed, idx32)
